# Initial kernel scaffold; baseline (speedup 1.0000x reference)
import jax
import jax.numpy as jnp
from jax.experimental import pallas as pl


def kernel(x, edge_index, feat, W_l1, W_r1, b1, W_l2, W_r2, b2, Wc1, bc1, Wc2, bc2):
    raise NotImplementedError("write your pallas kernel here")


# speedup vs baseline: 3.6356x; 3.6356x over previous
import functools

import jax
import jax.numpy as jnp
from jax import lax
from jax.experimental import pallas as pl
from jax.experimental.pallas import tpu as pltpu
from jax.experimental.pallas import tpu_sc as plsc

N = 10000
NP = 10240
E = 320000
D = 128
H = 128
F = 16

NC = 2
NS = 16
NW = NC * NS
EPW = E // NW
ECHUNK = 80
EITERS = EPW // ECHUNK
NPS = NP // NS
ZROWS = 128
DEGW = 128

_mesh = plsc.VectorSubcoreMesh(core_axis_name="c", subcore_axis_name="s")


def _seg_sum_body(y_hbm, src_hbm, dst_hbm, out_hbm,
                  idxs_v, idxd_v, rows_v, zbuf_v, acc_sh, sem):
    c = lax.axis_index("c")
    s = lax.axis_index("s")
    wid = s * NC + c

    zeros16 = jnp.zeros((16,), jnp.float32)

    def zfill(i, carry):
        r = i // (H // 16)
        k = i % (H // 16)
        zbuf_v[r, pl.ds(k * 16, 16)] = zeros16
        return carry
    lax.fori_loop(0, ZROWS * (H // 16), zfill, 0)

    def zcopy(j, carry):
        pltpu.sync_copy(zbuf_v, acc_sh.at[pl.ds(s * NPS + j * ZROWS, ZROWS)])
        return carry
    lax.fori_loop(0, NPS // ZROWS, zcopy, 0)

    plsc.subcore_barrier()

    base_e = wid * EPW

    def edge_body(i, carry):
        off = base_e + i * ECHUNK
        pltpu.sync_copy(src_hbm.at[pl.ds(off, ECHUNK)], idxs_v.at[0])
        pltpu.sync_copy(dst_hbm.at[pl.ds(off, ECHUNK)], idxd_v.at[0])
        pltpu.async_copy(y_hbm.at[idxs_v.at[0]], rows_v, sem).wait()
        pltpu.sync_copy(rows_v, acc_sh.at[idxd_v.at[0]], add=True)
        return carry
    lax.fori_loop(0, EITERS, edge_body, 0)

    plsc.subcore_barrier()

    pltpu.sync_copy(acc_sh.at[pl.ds(s * NPS, NPS)],
                    out_hbm.at[c, pl.ds(s * NPS, NPS)])


_seg_sum = pl.kernel(
    _seg_sum_body,
    out_type=jax.ShapeDtypeStruct((NC, NP, H), jnp.float32),
    mesh=_mesh,
    scratch_types=[
        pltpu.VMEM((1, ECHUNK), jnp.int32),
        pltpu.VMEM((1, ECHUNK), jnp.int32),
        pltpu.VMEM((ECHUNK, H), jnp.float32),
        pltpu.VMEM((ZROWS, H), jnp.float32),
        pltpu.VMEM_SHARED((NP, H), jnp.float32),
        pltpu.SemaphoreType.DMA,
    ],
    name="sc_seg_sum",
)


def _deg_body(dst_hbm, deg_hbm, idxd_v, ones_v, zdeg_v, dacc_sh):
    c = lax.axis_index("c")
    s = lax.axis_index("s")
    wid = s * NC + c

    zeros16 = jnp.zeros((16,), jnp.float32)
    ones16 = jnp.ones((16,), jnp.float32)

    def ofill(i, carry):
        r = i // (DEGW // 16)
        k = i % (DEGW // 16)
        ones_v[r, pl.ds(k * 16, 16)] = ones16
        return carry
    lax.fori_loop(0, ECHUNK * (DEGW // 16), ofill, 0)

    def zdfill(i, carry):
        r = i // (DEGW // 16)
        k = i % (DEGW // 16)
        zdeg_v[r, pl.ds(k * 16, 16)] = zeros16
        return carry
    lax.fori_loop(0, ZROWS * (DEGW // 16), zdfill, 0)

    def zcopy(j, carry):
        pltpu.sync_copy(zdeg_v, dacc_sh.at[pl.ds(s * NPS + j * ZROWS, ZROWS)])
        return carry
    lax.fori_loop(0, NPS // ZROWS, zcopy, 0)

    plsc.subcore_barrier()

    base_e = wid * EPW

    def edge_body(i, carry):
        off = base_e + i * ECHUNK
        pltpu.sync_copy(dst_hbm.at[pl.ds(off, ECHUNK)], idxd_v.at[0])
        pltpu.sync_copy(ones_v, dacc_sh.at[idxd_v.at[0]], add=True)
        return carry
    lax.fori_loop(0, EITERS, edge_body, 0)

    plsc.subcore_barrier()

    pltpu.sync_copy(dacc_sh.at[pl.ds(s * NPS, NPS)],
                    deg_hbm.at[c, pl.ds(s * NPS, NPS)])


_deg_count = pl.kernel(
    _deg_body,
    out_type=jax.ShapeDtypeStruct((NC, NP, DEGW), jnp.float32),
    mesh=_mesh,
    scratch_types=[
        pltpu.VMEM((1, ECHUNK), jnp.int32),
        pltpu.VMEM((ECHUNK, DEGW), jnp.float32),
        pltpu.VMEM((ZROWS, DEGW), jnp.float32),
        pltpu.VMEM_SHARED((NP, DEGW), jnp.float32),
    ],
    name="sc_deg_count",
)


def _edge_gather_body(p_hbm, q_hbm, src_hbm, dst_hbm, g_hbm,
                      idxs_v, idxd_v, pbuf_v, qbuf_v, sem1, sem2):
    c = lax.axis_index("c")
    s = lax.axis_index("s")
    wid = s * NC + c
    base_e = wid * EPW

    def edge_body(i, carry):
        off = base_e + i * ECHUNK
        pltpu.sync_copy(src_hbm.at[pl.ds(off, ECHUNK)], idxs_v.at[0])
        pltpu.sync_copy(dst_hbm.at[pl.ds(off, ECHUNK)], idxd_v.at[0])
        cp1 = pltpu.async_copy(p_hbm.at[idxs_v.at[0]], pbuf_v, sem1)
        cp2 = pltpu.async_copy(q_hbm.at[idxd_v.at[0]], qbuf_v, sem2)
        cp1.wait()
        cp2.wait()

        def add_body(r, carry2):
            for k in range(H // 16):
                sl = pl.ds(k * 16, 16)
                pbuf_v[r, sl] = pbuf_v[r, sl] + qbuf_v[r, sl]
            return carry2
        lax.fori_loop(0, ECHUNK, add_body, 0)
        pltpu.sync_copy(pbuf_v, g_hbm.at[pl.ds(off, ECHUNK)])
        return carry
    lax.fori_loop(0, EITERS, edge_body, 0)


_edge_gather = pl.kernel(
    _edge_gather_body,
    out_type=jax.ShapeDtypeStruct((E, H), jnp.float32),
    mesh=_mesh,
    scratch_types=[
        pltpu.VMEM((1, ECHUNK), jnp.int32),
        pltpu.VMEM((1, ECHUNK), jnp.int32),
        pltpu.VMEM((ECHUNK, H), jnp.float32),
        pltpu.VMEM((ECHUNK, H), jnp.float32),
        pltpu.SemaphoreType.DMA,
        pltpu.SemaphoreType.DMA,
    ],
    name="sc_edge_gather",
)


NB = 2048
EB = 2000


def _tc_in_proj_body(x_ref, wl_ref, wr_ref, b_ref, y_ref, r_ref):
    xv = x_ref[...]
    y_ref[...] = jnp.dot(xv, wl_ref[...], preferred_element_type=jnp.float32)
    r_ref[...] = (jnp.dot(xv, wr_ref[...], preferred_element_type=jnp.float32)
                  + b_ref[...])


def _tc_layer_body(p_ref, d_ref, r_ref, wl_ref, wr_ref, b_ref, y_ref, o_ref):
    acc = p_ref[0] + p_ref[1]
    deg = d_ref[0, :, 0:1] + d_ref[1, :, 0:1]
    h = jnp.maximum(acc / jnp.maximum(deg, 1.0) + r_ref[...], 0.0)
    y_ref[...] = jnp.dot(h, wl_ref[...], preferred_element_type=jnp.float32)
    o_ref[...] = (jnp.dot(h, wr_ref[...], preferred_element_type=jnp.float32)
                  + b_ref[...])


def _tc_classifier_body(g_ref, f_ref, c_ref, w2_ref, b2_ref, o_ref):
    t = jnp.maximum(
        g_ref[...] + jnp.dot(f_ref[...], c_ref[...],
                             preferred_element_type=jnp.float32),
        0.0)
    o_ref[...] = jnp.dot(t, w2_ref[...],
                         preferred_element_type=jnp.float32) + b2_ref[0, 0]


def _full(shape):
    return pl.BlockSpec(shape, lambda i: tuple(0 for _ in shape))


def _tc_in_proj(x, wl, wr, b):
    return pl.pallas_call(
        _tc_in_proj_body,
        grid=(NP // NB,),
        in_specs=[
            pl.BlockSpec((NB, D), lambda i: (i, 0)),
            _full((D, H)),
            _full((D, H)),
            _full((1, H)),
        ],
        out_specs=[
            pl.BlockSpec((NB, H), lambda i: (i, 0)),
            pl.BlockSpec((NB, H), lambda i: (i, 0)),
        ],
        out_shape=[
            jax.ShapeDtypeStruct((NP, H), jnp.float32),
            jax.ShapeDtypeStruct((NP, H), jnp.float32),
        ],
    )(x, wl, wr, b.reshape(1, H))


def _tc_layer(part, degp, r, wl, wr, b):
    return pl.pallas_call(
        _tc_layer_body,
        grid=(NP // NB,),
        in_specs=[
            pl.BlockSpec((NC, NB, H), lambda i: (0, i, 0)),
            pl.BlockSpec((NC, NB, DEGW), lambda i: (0, i, 0)),
            pl.BlockSpec((NB, H), lambda i: (i, 0)),
            _full((H, H)),
            _full((H, H)),
            _full((1, H)),
        ],
        out_specs=[
            pl.BlockSpec((NB, H), lambda i: (i, 0)),
            pl.BlockSpec((NB, H), lambda i: (i, 0)),
        ],
        out_shape=[
            jax.ShapeDtypeStruct((NP, H), jnp.float32),
            jax.ShapeDtypeStruct((NP, H), jnp.float32),
        ],
    )(part, degp, r, wl, wr, b.reshape(1, H))


def _tc_classifier(g, feat, cmat, w2, b2):
    return pl.pallas_call(
        _tc_classifier_body,
        grid=(E // EB,),
        in_specs=[
            pl.BlockSpec((EB, H), lambda i: (i, 0)),
            pl.BlockSpec((EB, F), lambda i: (i, 0)),
            _full((F, H)),
            _full((H, 1)),
            _full((1, 1)),
        ],
        out_specs=pl.BlockSpec((EB, 1), lambda i: (i, 0)),
        out_shape=jax.ShapeDtypeStruct((E, 1), jnp.float32),
    )(g, feat, cmat, w2, b2.reshape(1, 1))


def kernel(x, edge_index, feat, W_l1, W_r1, b1, W_l2, W_r2, b2,
           Wc1, bc1, Wc2, bc2):
    src = edge_index[0]
    dst = edge_index[1]
    xp = jnp.pad(x, ((0, NP - N), (0, 0)))

    y1, r1 = _tc_in_proj(xp, W_l1, W_r1, b1)
    degp = _deg_count(dst)
    part1 = _seg_sum(y1, src, dst)

    y2, r2 = _tc_layer(part1, degp, r1, W_l2, W_r2, b2)
    part2 = _seg_sum(y2, src, dst)

    A = Wc1[:H]
    B = Wc1[H:2 * H]
    C = Wc1[2 * H:]
    P, Q = _tc_layer(part2, degp, r2, A, B, bc1)

    g = _edge_gather(P, Q, src, dst)
    out = _tc_classifier(g, feat, C, Wc2, bc2)
    return out.reshape(-1)
